# baseline (speedup 1.0000x reference)
import functools

import jax
import jax.numpy as jnp
from jax import lax
from jax.experimental import pallas as pl
from jax.experimental.pallas import tpu as pltpu
from jax.experimental.pallas import tpu_sc as plsc

N = 10000
NP = 10240
D_IN = 128
H = 64
E = 320000
CH = 128
NCH = 2560
EP = NCH * CH
CPT = NCH // 32
RPT = NP // 16

NL = 100000
NLCH = 832
NLP = NLCH * CH
LCPT = NLCH // 32

_f32 = jnp.float32

_MESH = dict(core_axis_name="c", subcore_axis_name="s", num_cores=2,
             num_subcores=16)

_SC_PARAMS = pltpu.CompilerParams(use_tc_tiling_on_sc=False)
_SC_PARAMS_NLP = pltpu.CompilerParams(use_tc_tiling_on_sc=False,
                                      needs_layout_passes=False)


def _segsum_body(with_deg, *args):
    if with_deg:
        (table, src2d, dst2d, zacc, zdeg, acc_out, deg_out, sall, dall,
         rows4, ones_v, acc_sh, deg_sh,
         sg0, sg1, sg2, sg3, ss0, ss1, ss2, ss3, semd) = args
    else:
        (table, src2d, dst2d, zacc, acc_out, sall, dall, rows4, acc_sh,
         sg0, sg1, sg2, sg3, ss0, ss1, ss2, ss3) = args
        zdeg = deg_out = ones_v = deg_sh = semd = None
    sgs = (sg0, sg1, sg2, sg3)
    sss = (ss0, ss1, ss2, ss3)
    c = lax.axis_index("c")
    s = lax.axis_index("s")
    r0 = s * RPT
    sl = pl.ds(r0, RPT)
    pltpu.sync_copy(zacc.at[sl], acc_sh.at[sl])
    base = (c * 16 + s) * CPT
    pltpu.sync_copy(src2d.at[pl.ds(base, CPT)], sall)
    pltpu.sync_copy(dst2d.at[pl.ds(base, CPT)], dall)
    if with_deg:
        pltpu.sync_copy(zdeg.at[sl], deg_sh.at[sl])
        for i in range(CH // 16):
            ones_v[pl.ds(i * 16, 16)] = jnp.full((16,), 1.0, _f32)
    plsc.subcore_barrier()

    def gath(j, b):
        pltpu.async_copy(table.at[sall.at[j]], rows4.at[b], sgs[b])

    def wait_g(b):
        pltpu.make_async_copy(
            table.at[pl.ds(0, CH)], rows4.at[b], sgs[b]).wait()

    def wait_s(b):
        pltpu.make_async_copy(
            rows4.at[b], acc_sh.at[pl.ds(0, CH)], sss[b]).wait()

    def scat(j, b):
        pltpu.async_copy(rows4.at[b], acc_sh.at[dall.at[j]], sss[b],
                         add=True)
        if with_deg:
            pltpu.async_copy(ones_v, deg_sh.at[dall.at[j]], semd, add=True)

    def wait_d1(_=None):
        if with_deg:
            pltpu.make_async_copy(
                ones_v, deg_sh.at[pl.ds(0, CH)], semd).wait()

    gath(0, 0)
    gath(1, 1)
    for j in range(4):
        wait_g(j)
        scat(j, j)
        if j >= 2:
            wait_s(j - 2)
        gath(j + 2, (j + 2) % 4)

    def step(jj, carry):
        for b in range(4):
            j = 4 * jj + b
            wait_g(b)
            scat(j, b)
            wait_s((b + 2) % 4)
            wait_d1()
            gath(jnp.minimum(j + 2, CPT - 1), (b + 2) % 4)
        return carry

    lax.fori_loop(1, CPT // 4, step, 0)
    wait_g(0)
    wait_g(1)
    wait_s(2)
    wait_s(3)
    if with_deg:
        for _ in range(4):
            wait_d1()
    plsc.subcore_barrier()
    pltpu.sync_copy(acc_sh.at[sl], acc_out.at[c, s])
    if with_deg:
        pltpu.sync_copy(deg_sh.at[sl], deg_out.at[c, s])


def _make_segsum(with_deg):
    mesh = plsc.VectorSubcoreMesh(**_MESH)
    acc_t = jax.ShapeDtypeStruct((2, 16, RPT, H), _f32)
    sems = [pltpu.SemaphoreType.DMA] * 8
    if with_deg:
        out_type = (acc_t, jax.ShapeDtypeStruct((2, 16, RPT), _f32))
        scratch = [
            pltpu.VMEM((CPT, CH), jnp.int32),
            pltpu.VMEM((CPT, CH), jnp.int32),
            pltpu.VMEM((4, CH, H), _f32),
            pltpu.VMEM((CH,), _f32),
            pltpu.VMEM_SHARED((NP, H), _f32),
            pltpu.VMEM_SHARED((NP,), _f32),
        ] + sems + [pltpu.SemaphoreType.DMA]
    else:
        out_type = acc_t
        scratch = [
            pltpu.VMEM((CPT, CH), jnp.int32),
            pltpu.VMEM((CPT, CH), jnp.int32),
            pltpu.VMEM((4, CH, H), _f32),
            pltpu.VMEM_SHARED((NP, H), _f32),
        ] + sems
    return pl.kernel(
        functools.partial(_segsum_body, with_deg),
        out_type=out_type,
        mesh=mesh,
        scratch_types=scratch,
        compiler_params=_SC_PARAMS,
    )


def _head_gather_body(za, zb, row2d, col2d, hid_out, rall, cal, ra2, rb2, ho2,
                      sa0, sa1, sb0, sb1, sw0, sw1):
    c = lax.axis_index("c")
    s = lax.axis_index("s")
    base = (c * 16 + s) * LCPT
    pltpu.sync_copy(row2d.at[pl.ds(base, LCPT)], rall)
    pltpu.sync_copy(col2d.at[pl.ds(base, LCPT)], cal)
    sas = (sa0, sa1)
    sbs = (sb0, sb1)
    sws = (sw0, sw1)

    def gath(j, b):
        pltpu.async_copy(za.at[rall.at[j]], ra2.at[b], sas[b])
        pltpu.async_copy(zb.at[cal.at[j]], rb2.at[b], sbs[b])

    def wait_g(b):
        pltpu.make_async_copy(za.at[pl.ds(0, CH)], ra2.at[b], sas[b]).wait()
        pltpu.make_async_copy(zb.at[pl.ds(0, CH)], rb2.at[b], sbs[b]).wait()

    def wait_w(b):
        pltpu.make_async_copy(
            ho2.at[b], hid_out.at[base], sws[b]).wait()

    def compute(b):
        def addrow(i, cc):
            for k in range(H // 16):
                a = pl.ds(k * 16, 16)
                va = ra2[b, 2 * i, a] + rb2[b, 2 * i, a]
                vb = ra2[b, 2 * i + 1, a] + rb2[b, 2 * i + 1, a]
                ho2[b, i, pl.ds(k * 32, 32)] = plsc.pack(
                    va, vb, format=plsc.PackFormat.INTERLEAVED)
            return cc

        lax.fori_loop(0, CH // 2, addrow, 0)

    gath(0, 0)
    gath(1, 1)
    for b in range(2):
        wait_g(b)
        compute(b)
        gath(b + 2, b)
        pltpu.async_copy(ho2.at[b], hid_out.at[base + b], sws[b])

    def chunk(jj, carry):
        for b in range(2):
            j = 2 * jj + b
            wait_g(b)
            wait_w(b)
            compute(b)
            gath(jnp.minimum(j + 2, LCPT - 1), b)
            pltpu.async_copy(ho2.at[b], hid_out.at[base + j], sws[b])
        return carry

    lax.fori_loop(1, LCPT // 2, chunk, 0)
    for b in range(2):
        wait_g(b)
        wait_w(b)


def _make_head_gather():
    mesh = plsc.VectorSubcoreMesh(**_MESH)
    return pl.kernel(
        _head_gather_body,
        out_type=jax.ShapeDtypeStruct((NLCH, CH // 2, 2 * H), jnp.bfloat16),
        mesh=mesh,
        scratch_types=[
            pltpu.VMEM((LCPT, CH), jnp.int32),
            pltpu.VMEM((LCPT, CH), jnp.int32),
            pltpu.VMEM((2, CH, H), _f32),
            pltpu.VMEM((2, CH, H), _f32),
            pltpu.VMEM((2, CH // 2, 2 * H), jnp.bfloat16),
            pltpu.SemaphoreType.DMA,
            pltpu.SemaphoreType.DMA,
            pltpu.SemaphoreType.DMA,
            pltpu.SemaphoreType.DMA,
            pltpu.SemaphoreType.DMA,
            pltpu.SemaphoreType.DMA,
        ],
        compiler_params=_SC_PARAMS_NLP,
    )


def _proj_body(x_ref, wl_ref, wr_ref, p_ref, r_ref):
    xv = x_ref[...]
    p_ref[...] = jnp.dot(xv, wl_ref[...], preferred_element_type=_f32)
    r_ref[...] = jnp.dot(xv, wr_ref[...], preferred_element_type=_f32)


def _comb_body(relu, acc_ref, deg_ref, r_ref, b_ref, wl_ref, wr_ref, ob_ref,
               p_ref, rr_ref):
    accsum = acc_ref[0] + acc_ref[1]
    deg = jnp.maximum(deg_ref[0] + deg_ref[1], 1.0)
    hv = accsum / deg + r_ref[...] + b_ref[...]
    if relu:
        hv = jnp.maximum(hv, 0.0)
    ob = ob_ref[...]
    p_ref[...] = jnp.dot(hv, wl_ref[...], preferred_element_type=_f32) + ob
    rr_ref[...] = jnp.dot(hv, wr_ref[...], preferred_element_type=_f32) + ob


def _head_fin_body(hid_ref, w4_ref, b4_ref, o_ref):
    hv = jnp.maximum(hid_ref[...].astype(_f32), 0.0)
    t = jnp.dot(hv, w4_ref[...], preferred_element_type=_f32) + b4_ref[...]
    o_ref[...] = 1.0 / (1.0 + jnp.exp(-t))


def _tc_proj(x, wl, wr):
    nrow = x.shape[0]
    return pl.pallas_call(
        _proj_body,
        out_shape=(jax.ShapeDtypeStruct((nrow, H), _f32),
                   jax.ShapeDtypeStruct((nrow, H), _f32)),
    )(x, wl, wr)


def _tc_comb(acc, deg, r, b, wl, wr, ob, relu):
    return pl.pallas_call(
        functools.partial(_comb_body, relu),
        out_shape=(jax.ShapeDtypeStruct((NP, H), _f32),
                   jax.ShapeDtypeStruct((NP, H), _f32)),
    )(acc, deg, r, b, wl, wr, ob)


def _tc_head_fin(hid2, w42, b4):
    nrow = NLP // 2
    blk = nrow // 8
    return pl.pallas_call(
        _head_fin_body,
        grid=(8,),
        in_specs=[
            pl.BlockSpec((blk, 2 * H), lambda i: (i, 0)),
            pl.BlockSpec((2 * H, 2), lambda i: (0, 0)),
            pl.BlockSpec((1, 2), lambda i: (0, 0)),
        ],
        out_specs=pl.BlockSpec((blk, 2), lambda i: (i, 0)),
        out_shape=jax.ShapeDtypeStruct((nrow, 2), _f32),
    )(hid2, w42, b4)


def kernel(x, edge_index, edge_label_index, Wl1, Wr1, b1, Wl2, Wr2, b2,
           W3, b3, W4, b4):
    i32 = jnp.int32
    xp = jnp.concatenate([x, jnp.zeros((NP - N, D_IN), _f32)], axis=0)
    epad = N + jnp.arange(EP - E, dtype=i32) % (NP - N)
    src = jnp.concatenate([edge_index[0], epad])
    dst = jnp.concatenate([edge_index[1], epad])
    src2d = src.reshape(NCH, CH)
    dst2d = dst.reshape(NCH, CH)
    lpad = jnp.arange(NLP - NL, dtype=i32) % N
    row2d = jnp.concatenate([edge_label_index[0], lpad]).reshape(NLCH, CH)
    col2d = jnp.concatenate([edge_label_index[1], lpad]).reshape(NLCH, CH)
    zacc = jnp.zeros((NP, H), _f32)
    zdeg = jnp.zeros((NP,), _f32)

    p1, r1 = _tc_proj(xp, Wl1, Wr1)
    acc1, deg = _make_segsum(True)(p1, src2d, dst2d, zacc, zdeg)
    acc1 = acc1.reshape(2, NP, H)
    deg = deg.reshape(2, NP, 1)

    zb64 = jnp.zeros((1, H), _f32)
    p2, r2 = _tc_comb(acc1, deg, r1, b1.reshape(1, H), Wl2, Wr2, zb64,
                      relu=True)
    acc2 = _make_segsum(False)(p2, src2d, dst2d, zacc)
    acc2 = acc2.reshape(2, NP, H)

    w3a = W3[:H]
    w3b = W3[H:]
    hb3 = 0.5 * b3.reshape(1, H)
    za, zb = _tc_comb(acc2, deg, r2, b2.reshape(1, H), w3a, w3b, hb3,
                      relu=False)

    hid2 = _make_head_gather()(za, zb, row2d, col2d).reshape(NLP // 2, 2 * H)
    f = jnp.arange(H)
    idx0 = 32 * (f // 16) + 2 * (f % 16)
    w42 = jnp.zeros((2 * H, 2), _f32)
    w42 = w42.at[idx0, 0].set(W4[:, 0]).at[idx0 + 1, 1].set(W4[:, 0])
    out2 = _tc_head_fin(hid2, w42, jnp.broadcast_to(b4.reshape(1, 1), (1, 2)))
    return out2.reshape(NLP)[:NL]


# speedup vs baseline: 1.2161x; 1.2161x over previous
import functools

import jax
import jax.numpy as jnp
from jax import lax
from jax.experimental import pallas as pl
from jax.experimental.pallas import tpu as pltpu
from jax.experimental.pallas import tpu_sc as plsc

N = 10000
NP = 10240
D_IN = 128
H = 64
E = 320000
CH = 128
NCH = 2560
EP = NCH * CH
CPT = NCH // 32
RPT = NP // 16

NL = 100000
NLCH = 832
NLP = NLCH * CH
LCPT = NLCH // 32

_f32 = jnp.float32

_MESH = dict(core_axis_name="c", subcore_axis_name="s", num_cores=2,
             num_subcores=16)

_SC_PARAMS = pltpu.CompilerParams(use_tc_tiling_on_sc=False)


def _segsum_body(with_deg, *args):
    if with_deg:
        (table, src2d, dst2d, zacc, zdeg, acc_out, deg_out, sall, dall,
         rows4, ones_v, acc_sh, deg_sh,
         sg0, sg1, sg2, sg3, ss0, ss1, ss2, ss3, semd) = args
    else:
        (table, src2d, dst2d, zacc, acc_out, sall, dall, rows4, acc_sh,
         sg0, sg1, sg2, sg3, ss0, ss1, ss2, ss3) = args
        zdeg = deg_out = ones_v = deg_sh = semd = None
    sgs = (sg0, sg1, sg2, sg3)
    sss = (ss0, ss1, ss2, ss3)
    c = lax.axis_index("c")
    s = lax.axis_index("s")
    r0 = s * RPT
    sl = pl.ds(r0, RPT)
    pltpu.sync_copy(zacc.at[sl], acc_sh.at[sl])
    base = (c * 16 + s) * CPT
    pltpu.sync_copy(src2d.at[pl.ds(base, CPT)], sall)
    pltpu.sync_copy(dst2d.at[pl.ds(base, CPT)], dall)
    if with_deg:
        pltpu.sync_copy(zdeg.at[sl], deg_sh.at[sl])
        for i in range(CH // 16):
            ones_v[pl.ds(i * 16, 16)] = jnp.full((16,), 1.0, _f32)
    plsc.subcore_barrier()

    def gath(j, b):
        pltpu.async_copy(table.at[sall.at[j]], rows4.at[b], sgs[b])

    def wait_g(b):
        pltpu.make_async_copy(
            table.at[pl.ds(0, CH)], rows4.at[b], sgs[b]).wait()

    def wait_s(b):
        pltpu.make_async_copy(
            rows4.at[b], acc_sh.at[pl.ds(0, CH)], sss[b]).wait()

    def scat(j, b):
        pltpu.async_copy(rows4.at[b], acc_sh.at[dall.at[j]], sss[b],
                         add=True)
        if with_deg:
            pltpu.async_copy(ones_v, deg_sh.at[dall.at[j]], semd, add=True)

    def wait_d1(_=None):
        if with_deg:
            pltpu.make_async_copy(
                ones_v, deg_sh.at[pl.ds(0, CH)], semd).wait()

    gath(0, 0)
    gath(1, 1)
    for j in range(4):
        wait_g(j)
        scat(j, j)
        if j >= 2:
            wait_s(j - 2)
        gath(j + 2, (j + 2) % 4)

    def step(jj, carry):
        for b in range(4):
            j = 4 * jj + b
            wait_g(b)
            scat(j, b)
            wait_s((b + 2) % 4)
            wait_d1()
            gath(jnp.minimum(j + 2, CPT - 1), (b + 2) % 4)
        return carry

    lax.fori_loop(1, CPT // 4, step, 0)
    wait_g(0)
    wait_g(1)
    wait_s(2)
    wait_s(3)
    if with_deg:
        for _ in range(4):
            wait_d1()
    plsc.subcore_barrier()
    pltpu.sync_copy(acc_sh.at[sl], acc_out.at[c, s])
    if with_deg:
        pltpu.sync_copy(deg_sh.at[sl], deg_out.at[c, s])


def _make_segsum(with_deg):
    mesh = plsc.VectorSubcoreMesh(**_MESH)
    acc_t = jax.ShapeDtypeStruct((2, 16, RPT, H), _f32)
    sems = [pltpu.SemaphoreType.DMA] * 8
    if with_deg:
        out_type = (acc_t, jax.ShapeDtypeStruct((2, 16, RPT), _f32))
        scratch = [
            pltpu.VMEM((CPT, CH), jnp.int32),
            pltpu.VMEM((CPT, CH), jnp.int32),
            pltpu.VMEM((4, CH, H), _f32),
            pltpu.VMEM((CH,), _f32),
            pltpu.VMEM_SHARED((NP, H), _f32),
            pltpu.VMEM_SHARED((NP,), _f32),
        ] + sems + [pltpu.SemaphoreType.DMA]
    else:
        out_type = acc_t
        scratch = [
            pltpu.VMEM((CPT, CH), jnp.int32),
            pltpu.VMEM((CPT, CH), jnp.int32),
            pltpu.VMEM((4, CH, H), _f32),
            pltpu.VMEM_SHARED((NP, H), _f32),
        ] + sems
    return pl.kernel(
        functools.partial(_segsum_body, with_deg),
        out_type=out_type,
        mesh=mesh,
        scratch_types=scratch,
        compiler_params=_SC_PARAMS,
    )


def _head_gather_body(za, zb, row2d, col2d, hid_out, rall, cal, ra2, rb2, ho2,
                      sa0, sa1, sb0, sb1, sw0, sw1):
    c = lax.axis_index("c")
    s = lax.axis_index("s")
    base = (c * 16 + s) * LCPT
    pltpu.sync_copy(row2d.at[pl.ds(base, LCPT)], rall)
    pltpu.sync_copy(col2d.at[pl.ds(base, LCPT)], cal)
    sas = (sa0, sa1)
    sbs = (sb0, sb1)
    sws = (sw0, sw1)

    def gath(j, b):
        pltpu.async_copy(za.at[rall.at[j]], ra2.at[b], sas[b])
        pltpu.async_copy(zb.at[cal.at[j]], rb2.at[b], sbs[b])

    def wait_g(b):
        pltpu.make_async_copy(za.at[pl.ds(0, CH)], ra2.at[b], sas[b]).wait()
        pltpu.make_async_copy(zb.at[pl.ds(0, CH)], rb2.at[b], sbs[b]).wait()

    def wait_w(b):
        pltpu.make_async_copy(
            ho2.at[b], hid_out.at[base], sws[b]).wait()

    def compute(b):
        def addrow(i, cc):
            for k in range(H // 16):
                a = pl.ds(k * 16, 16)
                bsl = pl.ds(H + k * 16, 16)
                ho2[b, i, a] = ra2[b, 2 * i, a] + rb2[b, 2 * i, a]
                ho2[b, i, bsl] = ra2[b, 2 * i + 1, pl.ds(k * 16, 16)] + \
                    rb2[b, 2 * i + 1, pl.ds(k * 16, 16)]
            return cc

        lax.fori_loop(0, CH // 2, addrow, 0)

    gath(0, 0)
    gath(1, 1)
    for b in range(2):
        wait_g(b)
        compute(b)
        gath(b + 2, b)
        pltpu.async_copy(ho2.at[b], hid_out.at[base + b], sws[b])

    def chunk(jj, carry):
        for b in range(2):
            j = 2 * jj + b
            wait_g(b)
            wait_w(b)
            compute(b)
            gath(jnp.minimum(j + 2, LCPT - 1), b)
            pltpu.async_copy(ho2.at[b], hid_out.at[base + j], sws[b])
        return carry

    lax.fori_loop(1, LCPT // 2, chunk, 0)
    for b in range(2):
        wait_g(b)
        wait_w(b)


def _make_head_gather():
    mesh = plsc.VectorSubcoreMesh(**_MESH)
    return pl.kernel(
        _head_gather_body,
        out_type=jax.ShapeDtypeStruct((NLCH, CH // 2, 2 * H), _f32),
        mesh=mesh,
        scratch_types=[
            pltpu.VMEM((LCPT, CH), jnp.int32),
            pltpu.VMEM((LCPT, CH), jnp.int32),
            pltpu.VMEM((2, CH, H), _f32),
            pltpu.VMEM((2, CH, H), _f32),
            pltpu.VMEM((2, CH // 2, 2 * H), _f32),
            pltpu.SemaphoreType.DMA,
            pltpu.SemaphoreType.DMA,
            pltpu.SemaphoreType.DMA,
            pltpu.SemaphoreType.DMA,
            pltpu.SemaphoreType.DMA,
            pltpu.SemaphoreType.DMA,
        ],
        compiler_params=_SC_PARAMS,
    )


def _proj_body(x_ref, wl_ref, wr_ref, p_ref, r_ref):
    xv = x_ref[...]
    p_ref[...] = jnp.dot(xv, wl_ref[...], preferred_element_type=_f32)
    r_ref[...] = jnp.dot(xv, wr_ref[...], preferred_element_type=_f32)


def _comb_body(relu, acc_ref, deg_ref, r_ref, b_ref, wl_ref, wr_ref, ob_ref,
               p_ref, rr_ref):
    accsum = acc_ref[0] + acc_ref[1]
    deg = jnp.maximum(deg_ref[0] + deg_ref[1], 1.0)
    hv = accsum / deg + r_ref[...] + b_ref[...]
    if relu:
        hv = jnp.maximum(hv, 0.0)
    ob = ob_ref[...]
    p_ref[...] = jnp.dot(hv, wl_ref[...], preferred_element_type=_f32) + ob
    rr_ref[...] = jnp.dot(hv, wr_ref[...], preferred_element_type=_f32) + ob


def _head_fin_body(hid_ref, w4_ref, b4_ref, o_ref):
    hv = jnp.maximum(hid_ref[...], 0.0)
    t = jnp.dot(hv, w4_ref[...], preferred_element_type=_f32) + b4_ref[...]
    o_ref[...] = 1.0 / (1.0 + jnp.exp(-t))


def _tc_proj(x, wl, wr):
    nrow = x.shape[0]
    return pl.pallas_call(
        _proj_body,
        out_shape=(jax.ShapeDtypeStruct((nrow, H), _f32),
                   jax.ShapeDtypeStruct((nrow, H), _f32)),
    )(x, wl, wr)


def _tc_comb(acc, deg, r, b, wl, wr, ob, relu):
    return pl.pallas_call(
        functools.partial(_comb_body, relu),
        out_shape=(jax.ShapeDtypeStruct((NP, H), _f32),
                   jax.ShapeDtypeStruct((NP, H), _f32)),
    )(acc, deg, r, b, wl, wr, ob)


def _tc_head_fin(hid2, w42, b4):
    nrow = NLP // 2
    blk = nrow // 8
    return pl.pallas_call(
        _head_fin_body,
        grid=(8,),
        in_specs=[
            pl.BlockSpec((blk, 2 * H), lambda i: (i, 0)),
            pl.BlockSpec((2 * H, 2), lambda i: (0, 0)),
            pl.BlockSpec((1, 2), lambda i: (0, 0)),
        ],
        out_specs=pl.BlockSpec((blk, 2), lambda i: (i, 0)),
        out_shape=jax.ShapeDtypeStruct((nrow, 2), _f32),
    )(hid2, w42, b4)


def kernel(x, edge_index, edge_label_index, Wl1, Wr1, b1, Wl2, Wr2, b2,
           W3, b3, W4, b4):
    i32 = jnp.int32
    xp = jnp.concatenate([x, jnp.zeros((NP - N, D_IN), _f32)], axis=0)
    epad = N + jnp.arange(EP - E, dtype=i32) % (NP - N)
    src = jnp.concatenate([edge_index[0], epad])
    dst = jnp.concatenate([edge_index[1], epad])
    src2d = src.reshape(NCH, CH)
    dst2d = dst.reshape(NCH, CH)
    lpad = jnp.arange(NLP - NL, dtype=i32) % N
    row2d = jnp.concatenate([edge_label_index[0], lpad]).reshape(NLCH, CH)
    col2d = jnp.concatenate([edge_label_index[1], lpad]).reshape(NLCH, CH)
    zacc = jnp.zeros((NP, H), _f32)
    zdeg = jnp.zeros((NP,), _f32)

    p1, r1 = _tc_proj(xp, Wl1, Wr1)
    acc1, deg = _make_segsum(True)(p1, src2d, dst2d, zacc, zdeg)
    acc1 = acc1.reshape(2, NP, H)
    deg = deg.reshape(2, NP, 1)

    zb64 = jnp.zeros((1, H), _f32)
    p2, r2 = _tc_comb(acc1, deg, r1, b1.reshape(1, H), Wl2, Wr2, zb64,
                      relu=True)
    acc2 = _make_segsum(False)(p2, src2d, dst2d, zacc)
    acc2 = acc2.reshape(2, NP, H)

    w3a = W3[:H]
    w3b = W3[H:]
    hb3 = 0.5 * b3.reshape(1, H)
    za, zb = _tc_comb(acc2, deg, r2, b2.reshape(1, H), w3a, w3b, hb3,
                      relu=False)

    hid2 = _make_head_gather()(za, zb, row2d, col2d).reshape(NLP // 2, 2 * H)
    zcol = jnp.zeros((H, 1), _f32)
    w42 = jnp.concatenate(
        [jnp.concatenate([W4, zcol], axis=1),
         jnp.concatenate([zcol, W4], axis=1)], axis=0)
    out2 = _tc_head_fin(hid2, w42, jnp.broadcast_to(b4.reshape(1, 1), (1, 2)))
    return out2.reshape(NLP)[:NL]
